# baseline (speedup 1.0000x reference)
import jax
import jax.numpy as jnp
from jax import lax
from jax.experimental import pallas as pl

C_IN = 64
MEM = 128
NCLS = 27
MAPW = 128
H = 128
W = 128
M = MAPW * MAPW
EPS = 1e-5


def _gru_body(x_ref, h_ref, m_ref, wih_ref, whh_ref, bi_ref, bh_ref, o_ref):
    x = x_ref[...]
    h = h_ref[...]
    gi = jnp.dot(x, wih_ref[...], preferred_element_type=jnp.float32) + bi_ref[...]
    gh = jnp.dot(h, whh_ref[...], preferred_element_type=jnp.float32) + bh_ref[...]
    ir, iz, inn = gi[:, :MEM], gi[:, MEM:2 * MEM], gi[:, 2 * MEM:]
    hr, hz, hn = gh[:, :MEM], gh[:, MEM:2 * MEM], gh[:, 2 * MEM:]
    r = jax.nn.sigmoid(ir + hr)
    z = jax.nn.sigmoid(iz + hz)
    n = jnp.tanh(inn + r * hn)
    new = (1.0 - z) * n + z * h
    o_ref[...] = jnp.where(m_ref[...] > 0, new, h)


def _gru_update(gathered, state, m, wih_t, whh_t, bi, bh):
    BM = 2048
    grid = (M // BM,)
    return pl.pallas_call(
        _gru_body,
        grid=grid,
        in_specs=[
            pl.BlockSpec((BM, C_IN), lambda i: (i, 0)),
            pl.BlockSpec((BM, MEM), lambda i: (i, 0)),
            pl.BlockSpec((BM, 1), lambda i: (i, 0)),
            pl.BlockSpec((C_IN, 3 * MEM), lambda i: (0, 0)),
            pl.BlockSpec((MEM, 3 * MEM), lambda i: (0, 0)),
            pl.BlockSpec((1, 3 * MEM), lambda i: (0, 0)),
            pl.BlockSpec((1, 3 * MEM), lambda i: (0, 0)),
        ],
        out_specs=pl.BlockSpec((BM, MEM), lambda i: (i, 0)),
        out_shape=jax.ShapeDtypeStruct((M, MEM), jnp.float32),
    )(gathered, state, m, wih_t, whh_t, bi, bh)


def _make_conv_body(K, Wp, HWp, cout):
    def body(x_ref, w_ref, o_ref):
        def tap(k, acc):
            dy = k // K
            dx = k % K
            off = dy * Wp + dx
            xw = x_ref[pl.ds(off, HWp), :]
            wk = w_ref[pl.ds(k, 1), :, :][0]
            return acc + jnp.dot(xw, wk, preferred_element_type=jnp.float32)
        o_ref[...] = lax.fori_loop(0, K * K, tap,
                                   jnp.zeros((HWp, cout), jnp.float32))
    return body


def _make_bn_body(cnt, has_bn, relu):
    def body(a_ref, g_ref, b_ref, o_ref):
        acc = a_ref[...]
        if has_bn:
            inv = 1.0 / cnt
            mu = jnp.sum(acc, axis=0, keepdims=True) * inv
            d = acc - mu
            var = jnp.sum(d * d, axis=0, keepdims=True) * inv
            y = d * lax.rsqrt(var + EPS) * g_ref[...] + b_ref[...]
        else:
            y = acc + b_ref[...]
        o_ref[...] = jnp.maximum(y, 0.0) if relu else y
    return body


def _conv_bn(x, w, g, b, K, relu=True, has_bn=True):
    p = (K - 1) // 2
    Wp = W + 2 * p
    cout = w.shape[2]
    if p:
        Hp = H + 2 * p + 1
        xp = jnp.pad(x, ((p, p + 1), (p, p), (0, 0)))
        xf = xp.reshape(Hp * Wp, x.shape[2])
    else:
        xf = x.reshape(H * W, x.shape[2])
    HWp = H * Wp
    acc = pl.pallas_call(
        _make_conv_body(K, Wp, HWp, cout),
        out_shape=jax.ShapeDtypeStruct((HWp, cout), jnp.float32),
    )(xf, w)
    if p:
        val = acc.reshape(H, Wp, cout)[:, :W, :].reshape(H * W, cout)
    else:
        val = acc
    out = pl.pallas_call(
        _make_bn_body(float(H * W), has_bn, relu),
        out_shape=jax.ShapeDtypeStruct((H * W, cout), jnp.float32),
    )(val, g, b)
    return out.reshape(H, W, cout)


def _prep_w(w):
    kh, kw = w.shape[2], w.shape[3]
    return jnp.transpose(w, (2, 3, 1, 0)).reshape(kh * kw, w.shape[1], w.shape[0])


def kernel(features, proj_wtm, mask_outliers, heights, map_height, map_width, params):
    p = params
    N = features.shape[2] * features.shape[3]
    wih_t = jnp.transpose(p['gru_Wih'])
    whh_t = jnp.transpose(p['gru_Whh'])
    bi = p['gru_bih'].reshape(1, -1)
    bh = p['gru_bhh'].reshape(1, -1)

    state = jnp.zeros((M, MEM), jnp.float32)
    height_map = jnp.zeros((M,), jnp.float32)
    idx_n = jnp.arange(N, dtype=jnp.int32)
    for t in range(features.shape[0]):
        inl = (~mask_outliers[t]).reshape(-1)
        wtm = proj_wtm[t].astype(jnp.int32)
        x0 = wtm[:, :, 0].reshape(-1)
        y0 = jnp.clip(wtm[:, :, 1].reshape(-1), 0, map_height - 1)
        flat = (map_width * y0 + x0).astype(jnp.int32)
        h = jnp.where(inl, heights[t].reshape(-1) + 1000.0, -jnp.inf)
        seg_max = jax.ops.segment_max(h, flat, num_segments=M)
        m = (seg_max > height_map).astype(jnp.float32).reshape(M, 1)
        cand = jnp.where(h >= seg_max[flat], idx_n, jnp.int32(-1))
        arg = jnp.full((M,), -1, jnp.int32).at[flat].max(cand)
        height_map = jnp.maximum(height_map, seg_max)
        gathered = features[t].reshape(C_IN, N)[:, jnp.clip(arg, 0)].T
        state = _gru_update(gathered, state, m, wih_t, whh_t, bi, bh)

    mem = state.reshape(MAPW, MAPW, MEM)
    ones27 = jnp.ones((1, NCLS), jnp.float32)
    x = _conv_bn(mem, _prep_w(p['c1_w']), p['bn1_g'].reshape(1, -1), p['bn1_b'].reshape(1, -1), 7)
    x = _conv_bn(x, _prep_w(p['c2_w']), p['bn2_g'].reshape(1, -1), p['bn2_b'].reshape(1, -1), 3)
    x = _conv_bn(x, _prep_w(p['c3_w']), p['bn3_g'].reshape(1, -1), p['bn3_b'].reshape(1, -1), 3)
    y = _conv_bn(x, _prep_w(p['r1_w']), p['rbn1_g'].reshape(1, -1), p['rbn1_b'].reshape(1, -1), 1)
    y = _conv_bn(y, _prep_w(p['r2_w']), p['rbn2_g'].reshape(1, -1), p['rbn2_b'].reshape(1, -1), 3)
    y = _conv_bn(y, _prep_w(p['r3_w']), p['rbn3_g'].reshape(1, -1), p['rbn3_b'].reshape(1, -1), 1, relu=False)
    x = jax.nn.relu(y + x)
    x = _conv_bn(x, _prep_w(p['o1_w']), p['obn1_g'].reshape(1, -1), p['obn1_b'].reshape(1, -1), 3)
    x = _conv_bn(x, _prep_w(p['o2_w']), ones27, p['o2_b'].reshape(1, -1), 1, relu=False, has_bn=False)
    return jnp.transpose(x, (2, 0, 1))[None]
